# baseline (speedup 1.0000x reference)
import functools

import numpy as np

import jax
import jax.numpy as jnp
from jax import lax
from jax.experimental import pallas as pl
from jax.experimental.pallas import tpu as pltpu
from jax.experimental.pallas import tpu_sc as plsc

_B, _C, _H, _W = 8, 768, 16, 16
_N = 8192
_NPTS = _B * _N
_NC, _NS, _L = 2, 16, 16
_NW = _NC * _NS
_PTS_PER_W = _NPTS // _NW
_CHUNK = 16
_NCHUNKS = _PTS_PER_W // _CHUNK
_CW = _C // 2

_DNUMS = lax.GatherDimensionNumbers(
    offset_dims=(), collapsed_slice_dims=(0,), start_index_map=(0,))


def _splat(v, pidx):
    return lax.gather(v, pidx[:, None], _DNUMS, (1,),
                      mode=lax.GatherScatterMode.PROMISE_IN_BOUNDS)


def _packw(w):
    bits = plsc.bitcast(w, jnp.int32)
    hi = jnp.bitwise_and(bits, jnp.full((_L,), -65536, jnp.int32))
    word = jnp.bitwise_or(hi, lax.shift_right_logical(bits, 16))
    return plsc.bitcast(word, jnp.bfloat16)


def _gat(v, idx):
    return lax.gather(v, idx[:, None], _DNUMS, (1,),
                      mode=lax.GatherScatterMode.PROMISE_IN_BOUNDS)


def _coords(pts_v, ci, row_base):
    v0 = pts_v[pl.ds(ci * 2 * _CHUNK, _L)]
    v1 = pts_v[pl.ds(ci * 2 * _CHUNK + _L, _L)]
    lane = lax.iota(jnp.int32, _L)
    e8 = (lane % 8) * 2
    lo = lane < 8
    gx = jnp.where(lo, _gat(v0, e8), _gat(v1, e8))
    gy = jnp.where(lo, _gat(v0, e8 + 1), _gat(v1, e8 + 1))
    x = (gx + 1.0) * ((_W - 1) * 0.5)
    y = (gy + 1.0) * ((_H - 1) * 0.5)
    xi = jnp.minimum(x.astype(jnp.int32), _W - 1)
    yi = jnp.minimum(y.astype(jnp.int32), _H - 1)
    fx = x - xi.astype(jnp.float32)
    fy = y - yi.astype(jnp.float32)
    x1 = jnp.minimum(xi + 1, _W - 1)
    y1 = jnp.minimum(yi + 1, _H - 1)
    r0 = row_base + yi * _W
    r1 = row_base + y1 * _W
    return (r0 + xi, r0 + x1, r1 + xi, r1 + x1), (fx, fy)


def _sc_body(t_hbm, pts_hbm, out_hbm,
             pts_v, idx0, idx1, rows0, rows1, out0, out1,
             sg0, sg1, so0, so1):
    cid = lax.axis_index("c")
    sid = lax.axis_index("s")
    wid = sid * _NC + cid
    base = wid * _PTS_PER_W
    row_base = (base // _N) * (_H * _W)

    pltpu.sync_copy(pts_hbm.at[pl.ds(2 * base, 2 * _PTS_PER_W)], pts_v)

    bufs = ((idx0, rows0, out0, sg0, so0), (idx1, rows1, out1, sg1, so1))

    def fire(ci, b):
        idx_v, rows_v, _, sg, _ = bufs[b]
        (i00, i01, i10, i11), _ = _coords(pts_v, ci, row_base)
        idx_v[pl.ds(0, _L)] = i00
        idx_v[pl.ds(16, _L)] = i01
        idx_v[pl.ds(32, _L)] = i10
        idx_v[pl.ds(48, _L)] = i11
        pltpu.async_copy(t_hbm.at[idx_v], rows_v, sg)

    def consume(ci, b):
        idx_v, rows_v, out_v, sg, so = bufs[b]
        pltpu.make_async_copy(t_hbm.at[idx_v], rows_v, sg).wait()
        _, (fx, fy) = _coords(pts_v, ci, row_base)
        gx0 = 1.0 - fx
        gy0 = 1.0 - fy
        w00 = gx0 * gy0
        w01 = fx * gy0
        w10 = gx0 * fy
        w11 = fx * fy

        @pl.when(ci >= 2)
        def _():
            pltpu.make_async_copy(
                out_v, out_hbm.at[pl.ds(0, _CHUNK * _CW)], so).wait()

        @plsc.parallel_loop(0, _CHUNK, unroll=2)
        def pt_body(p):
            pidx = jnp.full((_L,), 0, jnp.int32) + p
            w0 = _packw(_splat(w00, pidx))
            w1 = _packw(_splat(w01, pidx))
            w2 = _packw(_splat(w10, pidx))
            w3 = _packw(_splat(w11, pidx))
            for j in range(_C // 32):
                s = pl.ds(j * _L, _L)
                a0 = plsc.bitcast(rows_v[p, s], jnp.bfloat16) * w0
                a1 = plsc.bitcast(rows_v[16 + p, s], jnp.bfloat16) * w1
                a2 = plsc.bitcast(rows_v[32 + p, s], jnp.bfloat16) * w2
                a3 = plsc.bitcast(rows_v[48 + p, s], jnp.bfloat16) * w3
                a = (a0 + a1) + (a2 + a3)
                out_v[pl.ds(p * _CW + j * _L, _L)] = plsc.bitcast(a, jnp.int32)
        pltpu.async_copy(
            out_v, out_hbm.at[pl.ds((base + ci * _CHUNK) * _CW, _CHUNK * _CW)],
            so)

    fire(0, 0)
    fire(1, 1)

    @pl.loop(0, _NCHUNKS, step=2)
    def _(g):
        for b in range(2):
            ci = g + b
            consume(ci, b)

            @pl.when(ci + 2 < _NCHUNKS)
            def _():
                fire(ci + 2, b)

    for b in range(2):
        _, _, out_v, _, so = bufs[b]
        pltpu.make_async_copy(
            out_v, out_hbm.at[pl.ds(0, _CHUNK * _CW)], so).wait()


@jax.jit
def kernel(features, points):
    B, C, H, W = features.shape
    table = features.reshape(B, C, H * W).transpose(0, 2, 1)
    table = table.reshape(B * H * W, C)
    tb = table.astype(jnp.bfloat16).reshape(B * H * W, C // 2, 2)
    table = lax.bitcast_convert_type(tb, jnp.int32)
    pts_flat = points.reshape(-1)

    mesh = plsc.VectorSubcoreMesh(core_axis_name="c", subcore_axis_name="s")
    run = functools.partial(
        pl.kernel,
        out_type=jax.ShapeDtypeStruct((_NPTS * _CW,), jnp.int32),
        mesh=mesh,
        compiler_params=pltpu.CompilerParams(needs_layout_passes=False),
        scratch_types=[
            pltpu.VMEM((2 * _PTS_PER_W,), jnp.float32),
            pltpu.VMEM((4 * _L,), jnp.int32),
            pltpu.VMEM((4 * _L,), jnp.int32),
            pltpu.VMEM((4 * _L, _C // 2), jnp.int32),
            pltpu.VMEM((4 * _L, _C // 2), jnp.int32),
            pltpu.VMEM((_CHUNK * _CW,), jnp.int32),
            pltpu.VMEM((_CHUNK * _CW,), jnp.int32),
            pltpu.SemaphoreType.DMA,
            pltpu.SemaphoreType.DMA,
            pltpu.SemaphoreType.DMA,
            pltpu.SemaphoreType.DMA,
        ],
    )(_sc_body)
    out = run(table, pts_flat)
    out = lax.bitcast_convert_type(out.reshape(_NPTS, _CW), jnp.bfloat16)
    return out.reshape(B, _N, C).astype(jnp.float32)


# speedup vs baseline: 2.4127x; 2.4127x over previous
import functools

import numpy as np

import jax
import jax.numpy as jnp
from jax import lax
from jax.experimental import pallas as pl
from jax.experimental.pallas import tpu as pltpu
from jax.experimental.pallas import tpu_sc as plsc

_B, _C, _H, _W = 8, 768, 16, 16
_N = 8192
_NPTS = _B * _N
_NC, _NS, _L = 2, 16, 16
_NW = _NC * _NS
_PTS_PER_W = _NPTS // _NW
_CHUNK = 16
_NCHUNKS = _PTS_PER_W // _CHUNK
_CVECS = _C // _L

_DNUMS = lax.GatherDimensionNumbers(
    offset_dims=(), collapsed_slice_dims=(0,), start_index_map=(0,))


def _splat(v, pidx):
    return lax.gather(v, pidx[:, None], _DNUMS, (1,),
                      mode=lax.GatherScatterMode.PROMISE_IN_BOUNDS)


def _packw_bits(w):
    bits = plsc.bitcast(w, jnp.int32)
    hi = jnp.bitwise_and(bits, jnp.full((_L,), -65536, jnp.int32))
    return jnp.bitwise_or(hi, lax.shift_right_logical(bits, 16))


def _gat(v, idx):
    return lax.gather(v, idx[:, None], _DNUMS, (1,),
                      mode=lax.GatherScatterMode.PROMISE_IN_BOUNDS)


def _coords(pts_v, ci, row_base):
    v0 = pts_v[pl.ds(ci * 2 * _CHUNK, _L)]
    v1 = pts_v[pl.ds(ci * 2 * _CHUNK + _L, _L)]
    lane = lax.iota(jnp.int32, _L)
    e8 = (lane % 8) * 2
    lo = lane < 8
    gx = jnp.where(lo, _gat(v0, e8), _gat(v1, e8))
    gy = jnp.where(lo, _gat(v0, e8 + 1), _gat(v1, e8 + 1))
    x = (gx + 1.0) * ((_W - 1) * 0.5)
    y = (gy + 1.0) * ((_H - 1) * 0.5)
    xi = jnp.minimum(x.astype(jnp.int32), _W - 1)
    yi = jnp.minimum(y.astype(jnp.int32), _H - 1)
    fx = x - xi.astype(jnp.float32)
    fy = y - yi.astype(jnp.float32)
    x1 = jnp.minimum(xi + 1, _W - 1)
    y1 = jnp.minimum(yi + 1, _H - 1)
    r0 = row_base + yi * _W
    r1 = row_base + y1 * _W
    return (r0 + xi, r0 + x1, r1 + xi, r1 + x1), (fx, fy)


def _sc_body(t_hbm, pts_hbm, out_hbm,
             pts_v, idx0, idx1, rows0, rows1, out0, out1,
             sg0, sg1, so0, so1):
    cid = lax.axis_index("c")
    sid = lax.axis_index("s")
    wid = sid * _NC + cid
    base = wid * _PTS_PER_W
    row_base = (base // _N) * (_H * _W)

    pltpu.sync_copy(pts_hbm.at[pl.ds(2 * base, 2 * _PTS_PER_W)], pts_v)

    bufs = ((idx0, rows0, out0, sg0, so0), (idx1, rows1, out1, sg1, so1))

    def fire(ci, b):
        idx_v, rows_v, _, sg, _ = bufs[b]
        (i00, i01, i10, i11), _ = _coords(pts_v, ci, row_base)
        idx_v[pl.ds(0, _L)] = i00
        idx_v[pl.ds(16, _L)] = i01
        idx_v[pl.ds(32, _L)] = i10
        idx_v[pl.ds(48, _L)] = i11
        pltpu.async_copy(t_hbm.at[idx_v], rows_v, sg)

    def consume(ci, b):
        idx_v, rows_v, out_v, sg, so = bufs[b]
        pltpu.make_async_copy(t_hbm.at[idx_v], rows_v, sg).wait()
        _, (fx, fy) = _coords(pts_v, ci, row_base)
        gx0 = 1.0 - fx
        gy0 = 1.0 - fy
        w00 = _packw_bits(gx0 * gy0)
        w01 = _packw_bits(fx * gy0)
        w10 = _packw_bits(gx0 * fy)
        w11 = _packw_bits(fx * fy)

        @pl.when(ci >= 2)
        def _():
            pltpu.make_async_copy(
                out_v, out_hbm.at[pl.ds(0, _CHUNK * _C)], so).wait()

        hi_mask = jnp.full((_L,), -65536, jnp.int32)

        @plsc.parallel_loop(0, _CHUNK, unroll=4)
        def pt_body(p):
            pidx = jnp.full((_L,), 0, jnp.int32) + p
            w0 = plsc.bitcast(_gat(w00, pidx), jnp.bfloat16)
            w1 = plsc.bitcast(_gat(w01, pidx), jnp.bfloat16)
            w2 = plsc.bitcast(_gat(w10, pidx), jnp.bfloat16)
            w3 = plsc.bitcast(_gat(w11, pidx), jnp.bfloat16)
            for j in range(_C // 32):
                s = pl.ds(j * _L, _L)
                a0 = plsc.bitcast(rows_v[p, s], jnp.bfloat16) * w0
                a1 = plsc.bitcast(rows_v[16 + p, s], jnp.bfloat16) * w1
                a2 = plsc.bitcast(rows_v[32 + p, s], jnp.bfloat16) * w2
                a3 = plsc.bitcast(rows_v[48 + p, s], jnp.bfloat16) * w3
                a = (a0 + a1) + (a2 + a3)
                v = plsc.bitcast(a, jnp.int32)
                lo = plsc.bitcast(lax.shift_left(v, 16), jnp.float32)
                hi = plsc.bitcast(jnp.bitwise_and(v, hi_mask), jnp.float32)
                out_v[pl.ds(p * _C + j * 32, _L)] = lo
                out_v[pl.ds(p * _C + j * 32 + 16, _L)] = hi
        pltpu.async_copy(
            out_v, out_hbm.at[pl.ds((base + ci * _CHUNK) * _C, _CHUNK * _C)],
            so)

    fire(0, 0)
    fire(1, 1)

    @pl.loop(0, _NCHUNKS, step=2)
    def _(g):
        for b in range(2):
            ci = g + b
            consume(ci, b)

            @pl.when(ci + 2 < _NCHUNKS)
            def _():
                fire(ci + 2, b)

    for b in range(2):
        _, _, out_v, _, so = bufs[b]
        pltpu.make_async_copy(
            out_v, out_hbm.at[pl.ds(0, _CHUNK * _C)], so).wait()


@jax.jit
def kernel(features, points):
    B, C, H, W = features.shape
    table = features.reshape(B, C, H * W).transpose(0, 2, 1)
    table = table.reshape(B * H * W, C)
    pos = np.arange(C)
    chan = 32 * (pos // 32) + (pos % 32) // 2 + 16 * (pos % 2)
    tb = table[:, chan].astype(jnp.bfloat16).reshape(B * H * W, C // 2, 2)
    table = lax.bitcast_convert_type(tb, jnp.int32)
    pts_flat = points.reshape(-1)

    mesh = plsc.VectorSubcoreMesh(core_axis_name="c", subcore_axis_name="s")
    run = functools.partial(
        pl.kernel,
        out_type=jax.ShapeDtypeStruct((_NPTS * _C,), jnp.float32),
        mesh=mesh,
        compiler_params=pltpu.CompilerParams(needs_layout_passes=False),
        scratch_types=[
            pltpu.VMEM((2 * _PTS_PER_W,), jnp.float32),
            pltpu.VMEM((4 * _L,), jnp.int32),
            pltpu.VMEM((4 * _L,), jnp.int32),
            pltpu.VMEM((4 * _L, _C // 2), jnp.int32),
            pltpu.VMEM((4 * _L, _C // 2), jnp.int32),
            pltpu.VMEM((_CHUNK * _C,), jnp.float32),
            pltpu.VMEM((_CHUNK * _C,), jnp.float32),
            pltpu.SemaphoreType.DMA,
            pltpu.SemaphoreType.DMA,
            pltpu.SemaphoreType.DMA,
            pltpu.SemaphoreType.DMA,
        ],
    )(_sc_body)
    out = run(table, pts_flat)
    return out.reshape(B, _N, C)


# speedup vs baseline: 2.4576x; 1.0186x over previous
import functools

import numpy as np

import jax
import jax.numpy as jnp
from jax import lax
from jax.experimental import pallas as pl
from jax.experimental.pallas import tpu as pltpu
from jax.experimental.pallas import tpu_sc as plsc

_B, _C, _H, _W = 8, 768, 16, 16
_N = 8192
_NPTS = _B * _N
_NC, _NS, _L = 2, 16, 16
_NW = _NC * _NS
_PTS_PER_W = _NPTS // _NW
_CHUNK = 16
_NCHUNKS = _PTS_PER_W // _CHUNK
_CVECS = _C // _L

_DNUMS = lax.GatherDimensionNumbers(
    offset_dims=(), collapsed_slice_dims=(0,), start_index_map=(0,))


def _splat(v, pidx):
    return lax.gather(v, pidx[:, None], _DNUMS, (1,),
                      mode=lax.GatherScatterMode.PROMISE_IN_BOUNDS)


def _packw_bits(w):
    bits = plsc.bitcast(w, jnp.int32)
    hi = jnp.bitwise_and(bits, jnp.full((_L,), -65536, jnp.int32))
    return jnp.bitwise_or(hi, lax.shift_right_logical(bits, 16))


def _gat(v, idx):
    return lax.gather(v, idx[:, None], _DNUMS, (1,),
                      mode=lax.GatherScatterMode.PROMISE_IN_BOUNDS)


def _coords(pts_v, ci, row_base):
    v0 = pts_v[pl.ds(ci * 2 * _CHUNK, _L)]
    v1 = pts_v[pl.ds(ci * 2 * _CHUNK + _L, _L)]
    lane = lax.iota(jnp.int32, _L)
    e8 = (lane % 8) * 2
    lo = lane < 8
    gx = jnp.where(lo, _gat(v0, e8), _gat(v1, e8))
    gy = jnp.where(lo, _gat(v0, e8 + 1), _gat(v1, e8 + 1))
    x = (gx + 1.0) * ((_W - 1) * 0.5)
    y = (gy + 1.0) * ((_H - 1) * 0.5)
    xi = jnp.minimum(x.astype(jnp.int32), _W - 1)
    yi = jnp.minimum(y.astype(jnp.int32), _H - 1)
    fx = x - xi.astype(jnp.float32)
    fy = y - yi.astype(jnp.float32)
    x1 = jnp.minimum(xi + 1, _W - 1)
    y1 = jnp.minimum(yi + 1, _H - 1)
    r0 = row_base + yi * _W
    r1 = row_base + y1 * _W
    return (r0 + xi, r0 + x1, r1 + xi, r1 + x1), (fx, fy)


def _sc_body(t_hbm, pts_hbm, out_hbm,
             pts_v, idx0, idx1, rows0, rows1, out0, out1,
             sg0, sg1, so0, so1):
    cid = lax.axis_index("c")
    sid = lax.axis_index("s")
    wid = sid * _NC + cid
    base = wid * _PTS_PER_W
    row_base = (base // _N) * (_H * _W)

    pltpu.sync_copy(pts_hbm.at[pl.ds(2 * base, 2 * _PTS_PER_W)], pts_v)

    bufs = ((idx0, rows0, out0, sg0, so0), (idx1, rows1, out1, sg1, so1))

    def fire(ci, b):
        idx_v, rows_v, _, sg, _ = bufs[b]
        (i00, _, i10, _), _ = _coords(pts_v, ci, row_base)
        idx_v[pl.ds(0, _L)] = i00
        idx_v[pl.ds(16, _L)] = i10
        pltpu.async_copy(t_hbm.at[idx_v], rows_v, sg)

    def consume(ci, b):
        idx_v, rows_v, out_v, sg, so = bufs[b]
        pltpu.make_async_copy(t_hbm.at[idx_v], rows_v, sg).wait()
        _, (fx, fy) = _coords(pts_v, ci, row_base)
        gx0 = 1.0 - fx
        gy0 = 1.0 - fy
        w00 = _packw_bits(gx0 * gy0)
        w01 = _packw_bits(fx * gy0)
        w10 = _packw_bits(gx0 * fy)
        w11 = _packw_bits(fx * fy)

        @pl.when(ci >= 2)
        def _():
            pltpu.make_async_copy(
                out_v, out_hbm.at[pl.ds(0, _CHUNK * _C)], so).wait()

        hi_mask = jnp.full((_L,), -65536, jnp.int32)

        @plsc.parallel_loop(0, _CHUNK, unroll=4)
        def pt_body(p):
            pidx = jnp.full((_L,), 0, jnp.int32) + p
            w0 = plsc.bitcast(_gat(w00, pidx), jnp.bfloat16)
            w1 = plsc.bitcast(_gat(w01, pidx), jnp.bfloat16)
            w2 = plsc.bitcast(_gat(w10, pidx), jnp.bfloat16)
            w3 = plsc.bitcast(_gat(w11, pidx), jnp.bfloat16)
            for j in range(_C // 32):
                s = pl.ds(j * _L, _L)
                s1 = pl.ds(_C // 2 + j * _L, _L)
                a0 = plsc.bitcast(rows_v[p, s], jnp.bfloat16) * w0
                a1 = plsc.bitcast(rows_v[p, s1], jnp.bfloat16) * w1
                a2 = plsc.bitcast(rows_v[16 + p, s], jnp.bfloat16) * w2
                a3 = plsc.bitcast(rows_v[16 + p, s1], jnp.bfloat16) * w3
                a = (a0 + a1) + (a2 + a3)
                v = plsc.bitcast(a, jnp.int32)
                lo = plsc.bitcast(lax.shift_left(v, 16), jnp.float32)
                hi = plsc.bitcast(jnp.bitwise_and(v, hi_mask), jnp.float32)
                out_v[pl.ds(p * _C + j * 32, _L)] = lo
                out_v[pl.ds(p * _C + j * 32 + 16, _L)] = hi
        pltpu.async_copy(
            out_v, out_hbm.at[pl.ds((base + ci * _CHUNK) * _C, _CHUNK * _C)],
            so)

    fire(0, 0)
    fire(1, 1)

    @pl.loop(0, _NCHUNKS, step=2)
    def _(g):
        for b in range(2):
            ci = g + b
            consume(ci, b)

            @pl.when(ci + 2 < _NCHUNKS)
            def _():
                fire(ci + 2, b)

    for b in range(2):
        _, _, out_v, _, so = bufs[b]
        pltpu.make_async_copy(
            out_v, out_hbm.at[pl.ds(0, _CHUNK * _C)], so).wait()


@jax.jit
def kernel(features, points):
    B, C, H, W = features.shape
    table = features.reshape(B, C, H * W).transpose(0, 2, 1)
    table = table.reshape(B * H * W, C)
    pos = np.arange(C)
    chan = 32 * (pos // 32) + (pos % 32) // 2 + 16 * (pos % 2)
    tb = table[:, chan].astype(jnp.bfloat16).reshape(B * H * W, C // 2, 2)
    table = lax.bitcast_convert_type(tb, jnp.int32)
    table = jnp.concatenate([table, jnp.roll(table, -1, axis=0)], axis=1)
    pts_flat = points.reshape(-1)

    mesh = plsc.VectorSubcoreMesh(core_axis_name="c", subcore_axis_name="s")
    run = functools.partial(
        pl.kernel,
        out_type=jax.ShapeDtypeStruct((_NPTS * _C,), jnp.float32),
        mesh=mesh,
        compiler_params=pltpu.CompilerParams(needs_layout_passes=False),
        scratch_types=[
            pltpu.VMEM((2 * _PTS_PER_W,), jnp.float32),
            pltpu.VMEM((2 * _L,), jnp.int32),
            pltpu.VMEM((2 * _L,), jnp.int32),
            pltpu.VMEM((2 * _L, _C), jnp.int32),
            pltpu.VMEM((2 * _L, _C), jnp.int32),
            pltpu.VMEM((_CHUNK * _C,), jnp.float32),
            pltpu.VMEM((_CHUNK * _C,), jnp.float32),
            pltpu.SemaphoreType.DMA,
            pltpu.SemaphoreType.DMA,
            pltpu.SemaphoreType.DMA,
            pltpu.SemaphoreType.DMA,
        ],
    )(_sc_body)
    out = run(table, pts_flat)
    return out.reshape(B, _N, C)
